# baseline (speedup 1.0000x reference)
import functools

import jax
import jax.numpy as jnp
import numpy as np
from jax import lax
from jax.experimental import pallas as pl
from jax.experimental.pallas import tpu as pltpu
from jax.experimental.pallas import tpu_sc as plsc

_N = 10000
_E = 320000
_D = 128
_G = 128
_L = 64
_NL = 3

_NC = 2
_NS = 16
_NW = _NC * _NS
_CH = 80
_KJ = 128
_EPT = _CH * _KJ
_EPAD = _NW * _EPT
_PADN = _EPAD - _E
_RPT = 632
_NP = _NS * _RPT
_NTRASH = _NP - _N

_BN = float(1.0 / np.sqrt(1.0 + 1e-5))

_mesh = plsc.VectorSubcoreMesh(
    core_axis_name="c", subcore_axis_name="s", num_cores=_NC, num_subcores=_NS
)


_NRB = 4
_NQB = 6


@functools.partial(
    pl.kernel,
    out_type=jax.ShapeDtypeStruct((_NC, _N, _D), jnp.float32),
    mesh=_mesh,
    scratch_types=[
        pltpu.VMEM((_NQB, 2, _CH), jnp.int32),
        pltpu.VMEM((_NRB, _CH, _D), jnp.float32),
        pltpu.VMEM_SHARED((_NP, _D), jnp.float32),
        pltpu.SemaphoreType.DMA,
        pltpu.SemaphoreType.DMA,
        pltpu.SemaphoreType.DMA,
        pltpu.SemaphoreType.DMA,
        pltpu.SemaphoreType.DMA,
        pltpu.SemaphoreType.DMA,
    ],
)
def _sc_agg(h_hbm, edge_hbm, out_hbm, idx, rows, agg_sh, isem, gsem,
            ssem0, ssem1, ssem2, ssem3):
    ssems = (ssem0, ssem1, ssem2, ssem3)
    c = lax.axis_index("c")
    s = lax.axis_index("s")
    wid = c * _NS + s

    zv = jnp.zeros((16,), jnp.float32)

    def _zrow(i, carry):
        for j in range(_D // 16):
            rows[0, i, pl.ds(j * 16, 16)] = zv
        return carry

    lax.fori_loop(0, _CH, _zrow, 0)
    zbase = s * _RPT
    for k in range(_RPT // _CH):
        pltpu.sync_copy(rows.at[0], agg_sh.at[pl.ds(zbase + k * _CH, _CH)])
    rem = _RPT - (_RPT // _CH) * _CH
    pltpu.sync_copy(rows.at[0].at[pl.ds(0, rem)],
                    agg_sh.at[pl.ds(zbase + (_RPT // _CH) * _CH, rem)])
    plsc.subcore_barrier()

    def _chunk(j, rb, qj, wait_scatter, fetch_ahead):
        if wait_scatter:
            pltpu.make_async_copy(rows.at[rb], agg_sh.at[idx.at[qj, 1]],
                                  ssems[rb]).wait()
        if fetch_ahead:
            pltpu.async_copy(edge_hbm.at[wid, j + 2], idx.at[(qj + 2) % _NQB],
                             isem)
        pltpu.make_async_copy(edge_hbm.at[wid, j], idx.at[qj], isem).wait()
        pltpu.async_copy(h_hbm.at[idx.at[qj, 0]], rows.at[rb], gsem).wait()
        pltpu.async_copy(rows.at[rb], agg_sh.at[idx.at[qj, 1]], ssems[rb],
                         add=True)

    pltpu.async_copy(edge_hbm.at[wid, 0], idx.at[0], isem)
    pltpu.async_copy(edge_hbm.at[wid, 1], idx.at[1], isem)
    for j in range(4):
        _chunk(j, j % _NRB, j % _NQB, wait_scatter=False, fetch_ahead=True)

    _UNROLL = 12

    def _grp(kk, carry):
        for b in range(_UNROLL):
            j = kk * _UNROLL + b + 4
            _chunk(j, (b + 4) % _NRB, (b + 4) % _NQB,
                   wait_scatter=True, fetch_ahead=True)
        return carry

    _STEADY = _KJ - 4 - 16
    lax.fori_loop(0, _STEADY // _UNROLL, _grp, 0)

    for j in range(_KJ - 16, _KJ):
        _chunk(j, j % _NRB, j % _NQB,
               wait_scatter=True, fetch_ahead=(j + 2 < _KJ))
    for b in range(_NRB):
        pltpu.make_async_copy(rows.at[b], agg_sh.at[idx.at[0, 1]],
                              ssems[b]).wait()
    plsc.subcore_barrier()

    obase = s * _RPT

    @pl.when(s < _NS - 1)
    def _():
        pltpu.sync_copy(
            agg_sh.at[pl.ds(obase, _RPT)], out_hbm.at[c].at[pl.ds(obase, _RPT)]
        )

    @pl.when(s == _NS - 1)
    def _():
        last = _N - (_NS - 1) * _RPT
        pltpu.sync_copy(
            agg_sh.at[pl.ds(obase, last)], out_hbm.at[c].at[pl.ds(obase, last)]
        )


_NB = 10
_BM = _N // _NB


def _mlp_body(h_ref, p_ref, w1_ref, b1_ref, a1_ref, g1_ref, be1_ref,
              w2_ref, b2_ref, a2_ref, o_ref):
    z = h_ref[...] + p_ref[0] + p_ref[1]
    t = lax.dot_general(z, w1_ref[...], (((1,), (1,)), ((), ())),
                        preferred_element_type=jnp.float32)
    t = t + b1_ref[...]
    t = jnp.where(t >= 0, t, a1_ref[...] * t)
    t = (t * _BN) * g1_ref[...] + be1_ref[...]
    u = lax.dot_general(t, w2_ref[...], (((1,), (1,)), ((), ())),
                        preferred_element_type=jnp.float32)
    u = u + b2_ref[...]
    o_ref[...] = jnp.where(u >= 0, u, a2_ref[...] * u)


_row_spec = pl.BlockSpec((_BM, _D), lambda i: (i, 0))
_par_spec = pl.BlockSpec((_NC, _BM, _D), lambda i: (0, i, 0))
_w_spec = pl.BlockSpec((_D, _D), lambda i: (0, 0))
_v_spec = pl.BlockSpec((1, _D), lambda i: (0, 0))

_mlp_call = pl.pallas_call(
    _mlp_body,
    grid=(_NB,),
    in_specs=[_row_spec, _par_spec, _w_spec, _v_spec, _v_spec, _v_spec,
              _v_spec, _w_spec, _v_spec, _v_spec],
    out_specs=_row_spec,
    out_shape=jax.ShapeDtypeStruct((_N, _D), jnp.float32),
    compiler_params=pltpu.CompilerParams(dimension_semantics=("arbitrary",)),
)


def _fin_body(h_ref, p_ref, w1_ref, b1_ref, a1_ref, g1_ref, be1_ref,
              w2_ref, b2_ref, a2_ref, bat_ref, gf_ref, bf_ref, fw_ref, fb_ref,
              o_ref, acc_ref):
    i = pl.program_id(0)
    z = h_ref[...] + p_ref[0] + p_ref[1]
    t = lax.dot_general(z, w1_ref[...], (((1,), (1,)), ((), ())),
                        preferred_element_type=jnp.float32)
    t = t + b1_ref[...]
    t = jnp.where(t >= 0, t, a1_ref[...] * t)
    t = (t * _BN) * g1_ref[...] + be1_ref[...]
    u = lax.dot_general(t, w2_ref[...], (((1,), (1,)), ((), ())),
                        preferred_element_type=jnp.float32)
    u = u + b2_ref[...]
    u = jnp.where(u >= 0, u, a2_ref[...] * u)

    b = bat_ref[0]
    gi = lax.broadcasted_iota(jnp.int32, (_G, _BM), 0)
    m = (b == gi).astype(jnp.float32)

    @pl.when(i == 0)
    def _():
        acc_ref[...] = jnp.zeros_like(acc_ref)

    acc_ref[...] += jnp.dot(m, u, preferred_element_type=jnp.float32)

    @pl.when(i == _NB - 1)
    def _():
        pooled = (acc_ref[...] * _BN) * gf_ref[...] + bf_ref[...]
        o_ref[...] = lax.dot_general(
            pooled, fw_ref[...], (((1,), (1,)), ((), ())),
            preferred_element_type=jnp.float32) + fb_ref[...]


_fin_call = pl.pallas_call(
    _fin_body,
    grid=(_NB,),
    in_specs=[_row_spec, _par_spec, _w_spec, _v_spec, _v_spec, _v_spec,
              _v_spec, _w_spec, _v_spec, _v_spec,
              pl.BlockSpec((1, 1, _BM), lambda i: (i, 0, 0)),
              pl.BlockSpec((1, _D), lambda i: (0, 0)),
              pl.BlockSpec((1, _D), lambda i: (0, 0)),
              pl.BlockSpec((_L, _D), lambda i: (0, 0)),
              pl.BlockSpec((1, _L), lambda i: (0, 0))],
    out_specs=pl.BlockSpec((_G, _L), lambda i: (0, 0)),
    out_shape=jax.ShapeDtypeStruct((_G, _L), jnp.float32),
    scratch_shapes=[pltpu.VMEM((_G, _D), jnp.float32)],
    compiler_params=pltpu.CompilerParams(dimension_semantics=("arbitrary",)),
)


def kernel(x, edge_index, batch, W1, b1, a1, g1, be1, W2, b2, a2, gf, bf, fcW, fcb):
    src = edge_index[0]
    dst = edge_index[1]
    padi = jnp.arange(_PADN, dtype=jnp.int32)
    src_p = jnp.concatenate([src, padi % _N]).reshape(_NW, _KJ, _CH)
    dst_p = jnp.concatenate([dst, _N + (padi % _NTRASH)]).reshape(_NW, _KJ, _CH)
    edge3 = jnp.stack([src_p, dst_p], axis=2)

    bat3 = batch.reshape(_NB, 1, _BM)
    b1r = b1.reshape(_NL, 1, _D)
    a1r = jnp.broadcast_to(a1[:, None, None], (_NL, 1, _D))
    g1r = g1.reshape(_NL, 1, _D)
    be1r = be1.reshape(_NL, 1, _D)
    b2r = b2.reshape(_NL, 1, _D)
    a2r = jnp.broadcast_to(a2[:, None, None], (_NL, 1, _D))
    gfr = gf.reshape(1, _D)
    bfr = bf.reshape(1, _D)
    fbr = fcb.reshape(1, _L)

    h = x
    for i in range(_NL - 1):
        p = _sc_agg(h, edge3)
        h = _mlp_call(h, p, W1[i], b1r[i], a1r[i], g1r[i], be1r[i],
                      W2[i], b2r[i], a2r[i])
    i = _NL - 1
    p = _sc_agg(h, edge3)
    return _fin_call(h, p, W1[i], b1r[i], a1r[i], g1r[i], be1r[i],
                     W2[i], b2r[i], a2r[i], bat3, gfr, bfr, fcW, fbr)


# speedup vs baseline: 1.1927x; 1.1927x over previous
import functools

import jax
import jax.numpy as jnp
import numpy as np
from jax import lax
from jax.experimental import pallas as pl
from jax.experimental.pallas import tpu as pltpu
from jax.experimental.pallas import tpu_sc as plsc

_N = 10000
_E = 320000
_D = 128
_G = 128
_L = 64
_NL = 3

_NC = 2
_NS = 16
_NW = _NC * _NS
_CH = 128
_KJ = 80
_EPT = _CH * _KJ
_EPAD = _NW * _EPT
_PADN = _EPAD - _E
_RPT = 632
_RPTL = 528
_NP = (_NS - 1) * _RPT + _RPTL
_NTRASH = _NP - _N

_BN = float(1.0 / np.sqrt(1.0 + 1e-5))

_mesh = plsc.VectorSubcoreMesh(
    core_axis_name="c", subcore_axis_name="s", num_cores=_NC, num_subcores=_NS
)


_NRB = 3
_NQB = 5


@functools.partial(
    pl.kernel,
    out_type=jax.ShapeDtypeStruct((_NC, _N, _D), jnp.float32),
    mesh=_mesh,
    scratch_types=[
        pltpu.VMEM((_NQB, 2, _CH), jnp.int32),
        pltpu.VMEM((_NRB, _CH, _D), jnp.float32),
        pltpu.VMEM_SHARED((_NP, _D), jnp.float32),
        pltpu.SemaphoreType.DMA,
        pltpu.SemaphoreType.DMA,
        pltpu.SemaphoreType.DMA,
        pltpu.SemaphoreType.DMA,
        pltpu.SemaphoreType.DMA,
    ],
)
def _sc_agg(h_hbm, edge_hbm, out_hbm, idx, rows, agg_sh, isem, gsem,
            ssem0, ssem1, ssem2):
    ssems = (ssem0, ssem1, ssem2)
    c = lax.axis_index("c")
    s = lax.axis_index("s")
    wid = c * _NS + s

    zv = jnp.zeros((16,), jnp.float32)

    def _zrow(i, carry):
        for j in range(_D // 16):
            rows[0, i, pl.ds(j * 16, 16)] = zv
        return carry

    lax.fori_loop(0, _CH, _zrow, 0)
    zbase = s * _RPT
    for k in range(_RPT // _CH):
        pltpu.sync_copy(rows.at[0], agg_sh.at[pl.ds(zbase + k * _CH, _CH)])
    zrem = zbase + (_RPT // _CH) * _CH

    @pl.when(s < _NS - 1)
    def _():
        pltpu.sync_copy(rows.at[0].at[pl.ds(0, _RPT - 512)],
                        agg_sh.at[pl.ds(zrem, _RPT - 512)])

    @pl.when(s == _NS - 1)
    def _():
        pltpu.sync_copy(rows.at[0].at[pl.ds(0, _RPTL - 512)],
                        agg_sh.at[pl.ds(zrem, _RPTL - 512)])

    plsc.subcore_barrier()

    def _chunk(j, rb, qj, wait_scatter, fetch_ahead):
        if wait_scatter:
            pltpu.make_async_copy(rows.at[rb], agg_sh.at[idx.at[qj, 1]],
                                  ssems[rb]).wait()
        if fetch_ahead:
            pltpu.async_copy(edge_hbm.at[wid, j + 2], idx.at[(qj + 2) % _NQB],
                             isem)
        pltpu.make_async_copy(edge_hbm.at[wid, j], idx.at[qj], isem).wait()
        pltpu.async_copy(h_hbm.at[idx.at[qj, 0]], rows.at[rb], gsem).wait()
        pltpu.async_copy(rows.at[rb], agg_sh.at[idx.at[qj, 1]], ssems[rb],
                         add=True)

    pltpu.async_copy(edge_hbm.at[wid, 0], idx.at[0], isem)
    pltpu.async_copy(edge_hbm.at[wid, 1], idx.at[1], isem)
    for j in range(_NRB):
        _chunk(j, j % _NRB, j % _NQB, wait_scatter=False, fetch_ahead=True)

    _UNROLL = 15

    def _grp(kk, carry):
        for b in range(_UNROLL):
            j = kk * _UNROLL + b + _NRB
            _chunk(j, (b + _NRB) % _NRB, (b + _NRB) % _NQB,
                   wait_scatter=True, fetch_ahead=True)
        return carry

    _STEADY = _KJ - _NRB - 2
    lax.fori_loop(0, _STEADY // _UNROLL, _grp, 0)

    for j in range(_KJ - 2, _KJ):
        _chunk(j, j % _NRB, j % _NQB,
               wait_scatter=True, fetch_ahead=False)
    for b in range(_NRB):
        pltpu.make_async_copy(rows.at[b], agg_sh.at[idx.at[0, 1]],
                              ssems[b]).wait()
    plsc.subcore_barrier()

    obase = s * _RPT

    @pl.when(s < _NS - 1)
    def _():
        pltpu.sync_copy(
            agg_sh.at[pl.ds(obase, _RPT)], out_hbm.at[c].at[pl.ds(obase, _RPT)]
        )

    @pl.when(s == _NS - 1)
    def _():
        last = _N - (_NS - 1) * _RPT
        pltpu.sync_copy(
            agg_sh.at[pl.ds(obase, last)], out_hbm.at[c].at[pl.ds(obase, last)]
        )


_NB = 10
_BM = _N // _NB


def _mlp_body(h_ref, p_ref, w1_ref, b1_ref, a1_ref, g1_ref, be1_ref,
              w2_ref, b2_ref, a2_ref, o_ref):
    z = h_ref[...] + p_ref[0] + p_ref[1]
    t = lax.dot_general(z, w1_ref[...], (((1,), (1,)), ((), ())),
                        preferred_element_type=jnp.float32)
    t = t + b1_ref[...]
    t = jnp.where(t >= 0, t, a1_ref[...] * t)
    t = (t * _BN) * g1_ref[...] + be1_ref[...]
    u = lax.dot_general(t, w2_ref[...], (((1,), (1,)), ((), ())),
                        preferred_element_type=jnp.float32)
    u = u + b2_ref[...]
    o_ref[...] = jnp.where(u >= 0, u, a2_ref[...] * u)


_row_spec = pl.BlockSpec((_BM, _D), lambda i: (i, 0))
_par_spec = pl.BlockSpec((_NC, _BM, _D), lambda i: (0, i, 0))
_w_spec = pl.BlockSpec((_D, _D), lambda i: (0, 0))
_v_spec = pl.BlockSpec((1, _D), lambda i: (0, 0))

_mlp_call = pl.pallas_call(
    _mlp_body,
    grid=(_NB,),
    in_specs=[_row_spec, _par_spec, _w_spec, _v_spec, _v_spec, _v_spec,
              _v_spec, _w_spec, _v_spec, _v_spec],
    out_specs=_row_spec,
    out_shape=jax.ShapeDtypeStruct((_N, _D), jnp.float32),
    compiler_params=pltpu.CompilerParams(dimension_semantics=("arbitrary",)),
)


def _fin_body(h_ref, p_ref, w1_ref, b1_ref, a1_ref, g1_ref, be1_ref,
              w2_ref, b2_ref, a2_ref, bat_ref, gf_ref, bf_ref, fw_ref, fb_ref,
              o_ref, acc_ref):
    i = pl.program_id(0)
    z = h_ref[...] + p_ref[0] + p_ref[1]
    t = lax.dot_general(z, w1_ref[...], (((1,), (1,)), ((), ())),
                        preferred_element_type=jnp.float32)
    t = t + b1_ref[...]
    t = jnp.where(t >= 0, t, a1_ref[...] * t)
    t = (t * _BN) * g1_ref[...] + be1_ref[...]
    u = lax.dot_general(t, w2_ref[...], (((1,), (1,)), ((), ())),
                        preferred_element_type=jnp.float32)
    u = u + b2_ref[...]
    u = jnp.where(u >= 0, u, a2_ref[...] * u)

    b = bat_ref[0]
    gi = lax.broadcasted_iota(jnp.int32, (_G, _BM), 0)
    m = (b == gi).astype(jnp.float32)

    @pl.when(i == 0)
    def _():
        acc_ref[...] = jnp.zeros_like(acc_ref)

    acc_ref[...] += jnp.dot(m, u, preferred_element_type=jnp.float32)

    @pl.when(i == _NB - 1)
    def _():
        pooled = (acc_ref[...] * _BN) * gf_ref[...] + bf_ref[...]
        o_ref[...] = lax.dot_general(
            pooled, fw_ref[...], (((1,), (1,)), ((), ())),
            preferred_element_type=jnp.float32) + fb_ref[...]


_fin_call = pl.pallas_call(
    _fin_body,
    grid=(_NB,),
    in_specs=[_row_spec, _par_spec, _w_spec, _v_spec, _v_spec, _v_spec,
              _v_spec, _w_spec, _v_spec, _v_spec,
              pl.BlockSpec((1, 1, _BM), lambda i: (i, 0, 0)),
              pl.BlockSpec((1, _D), lambda i: (0, 0)),
              pl.BlockSpec((1, _D), lambda i: (0, 0)),
              pl.BlockSpec((_L, _D), lambda i: (0, 0)),
              pl.BlockSpec((1, _L), lambda i: (0, 0))],
    out_specs=pl.BlockSpec((_G, _L), lambda i: (0, 0)),
    out_shape=jax.ShapeDtypeStruct((_G, _L), jnp.float32),
    scratch_shapes=[pltpu.VMEM((_G, _D), jnp.float32)],
    compiler_params=pltpu.CompilerParams(dimension_semantics=("arbitrary",)),
)


def kernel(x, edge_index, batch, W1, b1, a1, g1, be1, W2, b2, a2, gf, bf, fcW, fcb):
    src = edge_index[0]
    dst = edge_index[1]
    padi = jnp.arange(_PADN, dtype=jnp.int32)
    src_p = jnp.concatenate([src, padi % _N]).reshape(_NW, _KJ, _CH)
    dst_p = jnp.concatenate([dst, _N + (padi % _NTRASH)]).reshape(_NW, _KJ, _CH)
    edge3 = jnp.stack([src_p, dst_p], axis=2)

    bat3 = batch.reshape(_NB, 1, _BM)
    b1r = b1.reshape(_NL, 1, _D)
    a1r = jnp.broadcast_to(a1[:, None, None], (_NL, 1, _D))
    g1r = g1.reshape(_NL, 1, _D)
    be1r = be1.reshape(_NL, 1, _D)
    b2r = b2.reshape(_NL, 1, _D)
    a2r = jnp.broadcast_to(a2[:, None, None], (_NL, 1, _D))
    gfr = gf.reshape(1, _D)
    bfr = bf.reshape(1, _D)
    fbr = fcb.reshape(1, _L)

    h = x
    for i in range(_NL - 1):
        p = _sc_agg(h, edge3)
        h = _mlp_call(h, p, W1[i], b1r[i], a1r[i], g1r[i], be1r[i],
                      W2[i], b2r[i], a2r[i])
    i = _NL - 1
    p = _sc_agg(h, edge3)
    return _fin_call(h, p, W1[i], b1r[i], a1r[i], g1r[i], be1r[i],
                     W2[i], b2r[i], a2r[i], bat3, gfr, bfr, fcW, fbr)


# speedup vs baseline: 1.2014x; 1.0073x over previous
import functools

import jax
import jax.numpy as jnp
import numpy as np
from jax import lax
from jax.experimental import pallas as pl
from jax.experimental.pallas import tpu as pltpu
from jax.experimental.pallas import tpu_sc as plsc

_N = 10000
_E = 320000
_D = 128
_G = 128
_L = 64
_NL = 3

_NC = 2
_NS = 16
_NW = _NC * _NS
_CH = 128
_KJ = 80
_EPT = _CH * _KJ
_EPAD = _NW * _EPT
_PADN = _EPAD - _E
_RPT = 632
_RPTL = 528
_NP = (_NS - 1) * _RPT + _RPTL
_NTRASH = _NP - _N

_BN = float(1.0 / np.sqrt(1.0 + 1e-5))

_mesh = plsc.VectorSubcoreMesh(
    core_axis_name="c", subcore_axis_name="s", num_cores=_NC, num_subcores=_NS
)


_NRB = 3
_NQB = 5


@functools.partial(
    pl.kernel,
    out_type=jax.ShapeDtypeStruct((_NC, _N, _D), jnp.float32),
    mesh=_mesh,
    scratch_types=[
        pltpu.VMEM((_NQB, 2, _CH), jnp.int32),
        pltpu.VMEM((_NRB, _CH, _D), jnp.float32),
        pltpu.VMEM_SHARED((_NP, _D), jnp.float32),
        pltpu.SemaphoreType.DMA,
        pltpu.SemaphoreType.DMA,
        pltpu.SemaphoreType.DMA,
        pltpu.SemaphoreType.DMA,
        pltpu.SemaphoreType.DMA,
    ],
)
def _sc_agg(h_hbm, edge_hbm, z_hbm, out_hbm, idx, rows, agg_sh, isem, gsem,
            ssem0, ssem1, ssem2):
    ssems = (ssem0, ssem1, ssem2)
    c = lax.axis_index("c")
    s = lax.axis_index("s")
    wid = c * _NS + s

    pltpu.async_copy(edge_hbm.at[wid, 0], idx.at[0], isem)
    pltpu.async_copy(edge_hbm.at[wid, 1], idx.at[1], isem)

    zbase = s * _RPT

    @pl.when(s < _NS - 1)
    def _():
        pltpu.sync_copy(z_hbm.at[pl.ds(zbase, _RPT)],
                        agg_sh.at[pl.ds(zbase, _RPT)])

    @pl.when(s == _NS - 1)
    def _():
        pltpu.sync_copy(z_hbm.at[pl.ds(zbase, _RPTL)],
                        agg_sh.at[pl.ds(zbase, _RPTL)])

    plsc.subcore_barrier()

    def _chunk(j, rb, qj, wait_scatter, fetch_ahead):
        if wait_scatter:
            pltpu.make_async_copy(rows.at[rb], agg_sh.at[idx.at[qj, 1]],
                                  ssems[rb]).wait()
        if fetch_ahead:
            pltpu.async_copy(edge_hbm.at[wid, j + 2], idx.at[(qj + 2) % _NQB],
                             isem)
        pltpu.make_async_copy(edge_hbm.at[wid, j], idx.at[qj], isem).wait()
        pltpu.async_copy(h_hbm.at[idx.at[qj, 0]], rows.at[rb], gsem).wait()
        pltpu.async_copy(rows.at[rb], agg_sh.at[idx.at[qj, 1]], ssems[rb],
                         add=True)

    for j in range(_NRB):
        _chunk(j, j % _NRB, j % _NQB, wait_scatter=False, fetch_ahead=True)

    _UNROLL = 15

    def _grp(kk, carry):
        for b in range(_UNROLL):
            j = kk * _UNROLL + b + _NRB
            _chunk(j, (b + _NRB) % _NRB, (b + _NRB) % _NQB,
                   wait_scatter=True, fetch_ahead=True)
        return carry

    _STEADY = _KJ - _NRB - 2
    lax.fori_loop(0, _STEADY // _UNROLL, _grp, 0)

    for j in range(_KJ - 2, _KJ):
        _chunk(j, j % _NRB, j % _NQB,
               wait_scatter=True, fetch_ahead=False)
    for b in range(_NRB):
        pltpu.make_async_copy(rows.at[b], agg_sh.at[idx.at[0, 1]],
                              ssems[b]).wait()
    plsc.subcore_barrier()

    obase = s * _RPT

    @pl.when(s < _NS - 1)
    def _():
        pltpu.sync_copy(
            agg_sh.at[pl.ds(obase, _RPT)], out_hbm.at[c].at[pl.ds(obase, _RPT)]
        )

    @pl.when(s == _NS - 1)
    def _():
        last = _N - (_NS - 1) * _RPT
        pltpu.sync_copy(
            agg_sh.at[pl.ds(obase, last)], out_hbm.at[c].at[pl.ds(obase, last)]
        )


_NB = 5
_BM = _N // _NB


def _mlp_body(h_ref, p_ref, w1_ref, b1_ref, a1_ref, g1_ref, be1_ref,
              w2_ref, b2_ref, a2_ref, o_ref):
    z = h_ref[...] + p_ref[0] + p_ref[1]
    t = lax.dot_general(z, w1_ref[...], (((1,), (1,)), ((), ())),
                        preferred_element_type=jnp.float32)
    t = t + b1_ref[...]
    t = jnp.where(t >= 0, t, a1_ref[...] * t)
    t = (t * _BN) * g1_ref[...] + be1_ref[...]
    u = lax.dot_general(t, w2_ref[...], (((1,), (1,)), ((), ())),
                        preferred_element_type=jnp.float32)
    u = u + b2_ref[...]
    o_ref[...] = jnp.where(u >= 0, u, a2_ref[...] * u)


_row_spec = pl.BlockSpec((_BM, _D), lambda i: (i, 0))
_par_spec = pl.BlockSpec((_NC, _BM, _D), lambda i: (0, i, 0))
_w_spec = pl.BlockSpec((_D, _D), lambda i: (0, 0))
_v_spec = pl.BlockSpec((1, _D), lambda i: (0, 0))

_mlp_call = pl.pallas_call(
    _mlp_body,
    grid=(_NB,),
    in_specs=[_row_spec, _par_spec, _w_spec, _v_spec, _v_spec, _v_spec,
              _v_spec, _w_spec, _v_spec, _v_spec],
    out_specs=_row_spec,
    out_shape=jax.ShapeDtypeStruct((_N, _D), jnp.float32),
    compiler_params=pltpu.CompilerParams(dimension_semantics=("arbitrary",)),
)


def _fin_body(h_ref, p_ref, w1_ref, b1_ref, a1_ref, g1_ref, be1_ref,
              w2_ref, b2_ref, a2_ref, bat_ref, gf_ref, bf_ref, fw_ref, fb_ref,
              o_ref, acc_ref):
    i = pl.program_id(0)
    z = h_ref[...] + p_ref[0] + p_ref[1]
    t = lax.dot_general(z, w1_ref[...], (((1,), (1,)), ((), ())),
                        preferred_element_type=jnp.float32)
    t = t + b1_ref[...]
    t = jnp.where(t >= 0, t, a1_ref[...] * t)
    t = (t * _BN) * g1_ref[...] + be1_ref[...]
    u = lax.dot_general(t, w2_ref[...], (((1,), (1,)), ((), ())),
                        preferred_element_type=jnp.float32)
    u = u + b2_ref[...]
    u = jnp.where(u >= 0, u, a2_ref[...] * u)

    b = bat_ref[0]
    gi = lax.broadcasted_iota(jnp.int32, (_G, _BM), 0)
    m = (b == gi).astype(jnp.float32)

    @pl.when(i == 0)
    def _():
        acc_ref[...] = jnp.zeros_like(acc_ref)

    acc_ref[...] += jnp.dot(m, u, preferred_element_type=jnp.float32)

    @pl.when(i == _NB - 1)
    def _():
        pooled = (acc_ref[...] * _BN) * gf_ref[...] + bf_ref[...]
        o_ref[...] = lax.dot_general(
            pooled, fw_ref[...], (((1,), (1,)), ((), ())),
            preferred_element_type=jnp.float32) + fb_ref[...]


_fin_call = pl.pallas_call(
    _fin_body,
    grid=(_NB,),
    in_specs=[_row_spec, _par_spec, _w_spec, _v_spec, _v_spec, _v_spec,
              _v_spec, _w_spec, _v_spec, _v_spec,
              pl.BlockSpec((1, 1, _BM), lambda i: (i, 0, 0)),
              pl.BlockSpec((1, _D), lambda i: (0, 0)),
              pl.BlockSpec((1, _D), lambda i: (0, 0)),
              pl.BlockSpec((_L, _D), lambda i: (0, 0)),
              pl.BlockSpec((1, _L), lambda i: (0, 0))],
    out_specs=pl.BlockSpec((_G, _L), lambda i: (0, 0)),
    out_shape=jax.ShapeDtypeStruct((_G, _L), jnp.float32),
    scratch_shapes=[pltpu.VMEM((_G, _D), jnp.float32)],
    compiler_params=pltpu.CompilerParams(dimension_semantics=("arbitrary",)),
)


def kernel(x, edge_index, batch, W1, b1, a1, g1, be1, W2, b2, a2, gf, bf, fcW, fcb):
    src = edge_index[0]
    dst = edge_index[1]
    padi = jnp.arange(_PADN, dtype=jnp.int32)
    src_p = jnp.concatenate([src, padi % _N]).reshape(_NW, _KJ, _CH)
    dst_p = jnp.concatenate([dst, _N + (padi % _NTRASH)]).reshape(_NW, _KJ, _CH)
    edge3 = jnp.stack([src_p, dst_p], axis=2)

    bat3 = batch.reshape(_NB, 1, _BM)
    b1r = b1.reshape(_NL, 1, _D)
    a1r = jnp.broadcast_to(a1[:, None, None], (_NL, 1, _D))
    g1r = g1.reshape(_NL, 1, _D)
    be1r = be1.reshape(_NL, 1, _D)
    b2r = b2.reshape(_NL, 1, _D)
    a2r = jnp.broadcast_to(a2[:, None, None], (_NL, 1, _D))
    gfr = gf.reshape(1, _D)
    bfr = bf.reshape(1, _D)
    fbr = fcb.reshape(1, _L)

    zer = jnp.zeros((_NP, _D), jnp.float32)
    h = x
    for i in range(_NL - 1):
        p = _sc_agg(h, edge3, zer)
        h = _mlp_call(h, p, W1[i], b1r[i], a1r[i], g1r[i], be1r[i],
                      W2[i], b2r[i], a2r[i])
    i = _NL - 1
    p = _sc_agg(h, edge3, zer)
    return _fin_call(h, p, W1[i], b1r[i], a1r[i], g1r[i], be1r[i],
                     W2[i], b2r[i], a2r[i], bat3, gfr, bfr, fcW, fbr)


# speedup vs baseline: 1.2403x; 1.0324x over previous
import functools

import jax
import jax.numpy as jnp
import numpy as np
from jax import lax
from jax.experimental import pallas as pl
from jax.experimental.pallas import tpu as pltpu
from jax.experimental.pallas import tpu_sc as plsc

_N = 10000
_E = 320000
_D = 128
_G = 128
_L = 64
_NL = 3

_NC = 2
_NS = 16
_NW = _NC * _NS
_CH = 128
_KJ = 80
_EPT = _CH * _KJ
_EPAD = _NW * _EPT
_PADN = _EPAD - _E
_RPT = 632
_RPTL = 528
_NP = (_NS - 1) * _RPT + _RPTL
_NTRASH = _NP - _N

_BN = float(1.0 / np.sqrt(1.0 + 1e-5))

_mesh = plsc.VectorSubcoreMesh(
    core_axis_name="c", subcore_axis_name="s", num_cores=_NC, num_subcores=_NS
)


_NRB = 3
_NQB = 5


@functools.partial(
    pl.kernel,
    out_type=jax.ShapeDtypeStruct((_NC, _N, _D), jnp.float32),
    mesh=_mesh,
    scratch_types=[
        pltpu.VMEM((_NQB, 2, _CH), jnp.int32),
        pltpu.VMEM((_NRB, _CH, _D), jnp.float32),
        pltpu.VMEM_SHARED((_NP, _D), jnp.float32),
        pltpu.SemaphoreType.DMA,
        pltpu.SemaphoreType.DMA,
        pltpu.SemaphoreType.DMA,
        pltpu.SemaphoreType.DMA,
        pltpu.SemaphoreType.DMA,
    ],
)
def _sc_agg(h_hbm, edge_hbm, z_hbm, out_hbm, idx, rows, agg_sh, isem, gsem,
            ssem0, ssem1, ssem2):
    ssems = (ssem0, ssem1, ssem2)
    c = lax.axis_index("c")
    s = lax.axis_index("s")
    wid = c * _NS + s
    ebase = wid * _EPT

    def _fetch(j, slot):
        pltpu.async_copy(edge_hbm.at[:, pl.ds(ebase + j * _CH, _CH)],
                         idx.at[slot], isem)

    _fetch(0, 0)
    _fetch(1, 1)

    zbase = s * _RPT

    @pl.when(s < _NS - 1)
    def _():
        pltpu.sync_copy(z_hbm.at[pl.ds(zbase, _RPT)],
                        agg_sh.at[pl.ds(zbase, _RPT)])

    @pl.when(s == _NS - 1)
    def _():
        pltpu.sync_copy(z_hbm.at[pl.ds(zbase, _RPTL)],
                        agg_sh.at[pl.ds(zbase, _RPTL)])

    plsc.subcore_barrier()

    def _chunk(j, rb, qj, wait_scatter, fetch_ahead):
        if wait_scatter:
            pltpu.make_async_copy(rows.at[rb], agg_sh.at[idx.at[qj, 1]],
                                  ssems[rb]).wait()
        if fetch_ahead:
            _fetch(j + 2, (qj + 2) % _NQB)
        pltpu.make_async_copy(edge_hbm.at[:, pl.ds(ebase, _CH)], idx.at[qj],
                              isem).wait()
        pltpu.async_copy(h_hbm.at[idx.at[qj, 0]], rows.at[rb], gsem).wait()
        pltpu.async_copy(rows.at[rb], agg_sh.at[idx.at[qj, 1]], ssems[rb],
                         add=True)

    for j in range(_NRB):
        _chunk(j, j % _NRB, j % _NQB, wait_scatter=False, fetch_ahead=True)

    _UNROLL = 15

    def _grp(kk, carry):
        for b in range(_UNROLL):
            j = kk * _UNROLL + b + _NRB
            _chunk(j, (b + _NRB) % _NRB, (b + _NRB) % _NQB,
                   wait_scatter=True, fetch_ahead=True)
        return carry

    ngrp = jnp.where(wid == _NW - 1, (20 - _NRB - 2) // _UNROLL,
                     (_KJ - _NRB - 2) // _UNROLL)
    lax.fori_loop(0, ngrp, _grp, 0)

    jlast = ngrp * _UNROLL + _NRB
    _chunk(jlast, 0, 3, wait_scatter=True, fetch_ahead=False)
    _chunk(jlast + 1, 1, 4, wait_scatter=True, fetch_ahead=False)
    for b in range(_NRB):
        pltpu.make_async_copy(rows.at[b], agg_sh.at[idx.at[0, 1]],
                              ssems[b]).wait()
    plsc.subcore_barrier()

    obase = s * _RPT

    @pl.when(s < _NS - 1)
    def _():
        pltpu.sync_copy(
            agg_sh.at[pl.ds(obase, _RPT)], out_hbm.at[c].at[pl.ds(obase, _RPT)]
        )

    @pl.when(s == _NS - 1)
    def _():
        last = _N - (_NS - 1) * _RPT
        pltpu.sync_copy(
            agg_sh.at[pl.ds(obase, last)], out_hbm.at[c].at[pl.ds(obase, last)]
        )


_NB = 5
_BM = _N // _NB


def _mlp_body(h_ref, p_ref, w1_ref, b1_ref, a1_ref, g1_ref, be1_ref,
              w2_ref, b2_ref, a2_ref, o_ref):
    z = h_ref[...] + p_ref[0] + p_ref[1]
    t = lax.dot_general(z, w1_ref[...], (((1,), (1,)), ((), ())),
                        preferred_element_type=jnp.float32)
    t = t + b1_ref[...]
    t = jnp.where(t >= 0, t, a1_ref[...] * t)
    t = (t * _BN) * g1_ref[...] + be1_ref[...]
    u = lax.dot_general(t, w2_ref[...], (((1,), (1,)), ((), ())),
                        preferred_element_type=jnp.float32)
    u = u + b2_ref[...]
    o_ref[...] = jnp.where(u >= 0, u, a2_ref[...] * u)


_row_spec = pl.BlockSpec((_BM, _D), lambda i: (i, 0))
_par_spec = pl.BlockSpec((_NC, _BM, _D), lambda i: (0, i, 0))
_w_spec = pl.BlockSpec((_D, _D), lambda i: (0, 0))
_v_spec = pl.BlockSpec((1, _D), lambda i: (0, 0))

_mlp_call = pl.pallas_call(
    _mlp_body,
    grid=(_NB,),
    in_specs=[_row_spec, _par_spec, _w_spec, _v_spec, _v_spec, _v_spec,
              _v_spec, _w_spec, _v_spec, _v_spec],
    out_specs=_row_spec,
    out_shape=jax.ShapeDtypeStruct((_N, _D), jnp.float32),
    compiler_params=pltpu.CompilerParams(dimension_semantics=("arbitrary",)),
)


def _fin_body(h_ref, p_ref, w1_ref, b1_ref, a1_ref, g1_ref, be1_ref,
              w2_ref, b2_ref, a2_ref, bat_ref, gf_ref, bf_ref, fw_ref, fb_ref,
              o_ref, acc_ref):
    i = pl.program_id(0)
    z = h_ref[...] + p_ref[0] + p_ref[1]
    t = lax.dot_general(z, w1_ref[...], (((1,), (1,)), ((), ())),
                        preferred_element_type=jnp.float32)
    t = t + b1_ref[...]
    t = jnp.where(t >= 0, t, a1_ref[...] * t)
    t = (t * _BN) * g1_ref[...] + be1_ref[...]
    u = lax.dot_general(t, w2_ref[...], (((1,), (1,)), ((), ())),
                        preferred_element_type=jnp.float32)
    u = u + b2_ref[...]
    u = jnp.where(u >= 0, u, a2_ref[...] * u)

    b = bat_ref[0]
    gi = lax.broadcasted_iota(jnp.int32, (_G, _BM), 0)
    m = (b == gi).astype(jnp.float32)

    @pl.when(i == 0)
    def _():
        acc_ref[...] = jnp.zeros_like(acc_ref)

    acc_ref[...] += jnp.dot(m, u, preferred_element_type=jnp.float32)

    @pl.when(i == _NB - 1)
    def _():
        pooled = (acc_ref[...] * _BN) * gf_ref[...] + bf_ref[...]
        o_ref[...] = lax.dot_general(
            pooled, fw_ref[...], (((1,), (1,)), ((), ())),
            preferred_element_type=jnp.float32) + fb_ref[...]


_fin_call = pl.pallas_call(
    _fin_body,
    grid=(_NB,),
    in_specs=[_row_spec, _par_spec, _w_spec, _v_spec, _v_spec, _v_spec,
              _v_spec, _w_spec, _v_spec, _v_spec,
              pl.BlockSpec((1, 1, _BM), lambda i: (i, 0, 0)),
              pl.BlockSpec((1, _D), lambda i: (0, 0)),
              pl.BlockSpec((1, _D), lambda i: (0, 0)),
              pl.BlockSpec((_L, _D), lambda i: (0, 0)),
              pl.BlockSpec((1, _L), lambda i: (0, 0))],
    out_specs=pl.BlockSpec((_G, _L), lambda i: (0, 0)),
    out_shape=jax.ShapeDtypeStruct((_G, _L), jnp.float32),
    scratch_shapes=[pltpu.VMEM((_G, _D), jnp.float32)],
    compiler_params=pltpu.CompilerParams(dimension_semantics=("arbitrary",)),
)


def kernel(x, edge_index, batch, W1, b1, a1, g1, be1, W2, b2, a2, gf, bf, fcW, fcb):
    bat3 = batch.reshape(_NB, 1, _BM)
    b1r = b1.reshape(_NL, 1, _D)
    a1r = jnp.broadcast_to(a1[:, None, None], (_NL, 1, _D))
    g1r = g1.reshape(_NL, 1, _D)
    be1r = be1.reshape(_NL, 1, _D)
    b2r = b2.reshape(_NL, 1, _D)
    a2r = jnp.broadcast_to(a2[:, None, None], (_NL, 1, _D))
    gfr = gf.reshape(1, _D)
    bfr = bf.reshape(1, _D)
    fbr = fcb.reshape(1, _L)

    zer = jnp.zeros((_NP, _D), jnp.float32)
    h = x
    for i in range(_NL - 1):
        p = _sc_agg(h, edge_index, zer)
        h = _mlp_call(h, p, W1[i], b1r[i], a1r[i], g1r[i], be1r[i],
                      W2[i], b2r[i], a2r[i])
    i = _NL - 1
    p = _sc_agg(h, edge_index, zer)
    return _fin_call(h, p, W1[i], b1r[i], a1r[i], g1r[i], be1r[i],
                     W2[i], b2r[i], a2r[i], bat3, gfr, bfr, fcW, fbr)


# speedup vs baseline: 1.2505x; 1.0082x over previous
import functools

import jax
import jax.numpy as jnp
import numpy as np
from jax import lax
from jax.experimental import pallas as pl
from jax.experimental.pallas import tpu as pltpu
from jax.experimental.pallas import tpu_sc as plsc

_N = 10000
_E = 320000
_D = 128
_G = 128
_L = 64
_NL = 3

_NC = 2
_NS = 16
_NW = _NC * _NS
_CH = 128
_KJ = 80
_EPT = _CH * _KJ
_EPAD = _NW * _EPT
_PADN = _EPAD - _E
_RPT = 632
_RPTL = 528
_NP = (_NS - 1) * _RPT + _RPTL
_NTRASH = _NP - _N

_BN = float(1.0 / np.sqrt(1.0 + 1e-5))

_mesh = plsc.VectorSubcoreMesh(
    core_axis_name="c", subcore_axis_name="s", num_cores=_NC, num_subcores=_NS
)


_NRB = 2
_NQB = 4


@functools.partial(
    pl.kernel,
    out_type=jax.ShapeDtypeStruct((_NC, _N, _D), jnp.float32),
    mesh=_mesh,
    scratch_types=[
        pltpu.VMEM((_NQB, 2, _CH), jnp.int32),
        pltpu.VMEM((_NRB, _CH, _D), jnp.float32),
        pltpu.VMEM_SHARED((_NP, _D), jnp.float32),
        pltpu.SemaphoreType.DMA,
        pltpu.SemaphoreType.DMA,
        pltpu.SemaphoreType.DMA,
        pltpu.SemaphoreType.DMA,
    ],
)
def _sc_agg(h_hbm, edge_hbm, z_hbm, out_hbm, idx, rows, agg_sh, isem, gsem,
            ssem0, ssem1):
    ssems = (ssem0, ssem1)
    c = lax.axis_index("c")
    s = lax.axis_index("s")
    wid = c * _NS + s
    ebase = wid * _EPT

    def _fetch(j, slot):
        pltpu.async_copy(edge_hbm.at[:, pl.ds(ebase + j * _CH, _CH)],
                         idx.at[slot], isem)

    _fetch(0, 0)
    _fetch(1, 1)

    zbase = s * _RPT

    @pl.when(s < _NS - 1)
    def _():
        pltpu.sync_copy(z_hbm.at[pl.ds(zbase, _RPT)],
                        agg_sh.at[pl.ds(zbase, _RPT)])

    @pl.when(s == _NS - 1)
    def _():
        pltpu.sync_copy(z_hbm.at[pl.ds(zbase, _RPTL)],
                        agg_sh.at[pl.ds(zbase, _RPTL)])

    plsc.subcore_barrier()

    def _chunk(j, rb, qj, wait_scatter, fetch_ahead):
        if wait_scatter:
            pltpu.make_async_copy(rows.at[rb], agg_sh.at[idx.at[qj, 1]],
                                  ssems[rb]).wait()
        if fetch_ahead:
            _fetch(j + 2, (qj + 2) % _NQB)
        pltpu.make_async_copy(edge_hbm.at[:, pl.ds(ebase, _CH)], idx.at[qj],
                              isem).wait()
        pltpu.async_copy(h_hbm.at[idx.at[qj, 0]], rows.at[rb], gsem).wait()
        pltpu.async_copy(rows.at[rb], agg_sh.at[idx.at[qj, 1]], ssems[rb],
                         add=True)

    for j in range(_NRB):
        _chunk(j, j % _NRB, j % _NQB, wait_scatter=False, fetch_ahead=True)

    _UNROLL = 4

    def _grp(kk, carry):
        for b in range(_UNROLL):
            j = kk * _UNROLL + b + _NRB
            _chunk(j, (b + _NRB) % _NRB, (b + _NRB) % _NQB,
                   wait_scatter=True, fetch_ahead=True)
        return carry

    ngrp = jnp.where(wid == _NW - 1, (20 - _NRB - 2) // _UNROLL,
                     (_KJ - _NRB - 2) // _UNROLL)
    lax.fori_loop(0, ngrp, _grp, 0)

    jlast = ngrp * _UNROLL + _NRB
    _chunk(jlast, 0, 2, wait_scatter=True, fetch_ahead=False)
    _chunk(jlast + 1, 1, 3, wait_scatter=True, fetch_ahead=False)
    for b in range(_NRB):
        pltpu.make_async_copy(rows.at[b], agg_sh.at[idx.at[0, 1]],
                              ssems[b]).wait()
    plsc.subcore_barrier()

    obase = s * _RPT

    @pl.when(s < _NS - 1)
    def _():
        pltpu.sync_copy(
            agg_sh.at[pl.ds(obase, _RPT)], out_hbm.at[c].at[pl.ds(obase, _RPT)]
        )

    @pl.when(s == _NS - 1)
    def _():
        last = _N - (_NS - 1) * _RPT
        pltpu.sync_copy(
            agg_sh.at[pl.ds(obase, last)], out_hbm.at[c].at[pl.ds(obase, last)]
        )


_NB = 5
_BM = _N // _NB


def _mlp_body(h_ref, p_ref, w1_ref, b1_ref, a1_ref, g1_ref, be1_ref,
              w2_ref, b2_ref, a2_ref, o_ref):
    z = h_ref[...] + p_ref[0] + p_ref[1]
    t = lax.dot_general(z, w1_ref[...], (((1,), (1,)), ((), ())),
                        preferred_element_type=jnp.float32)
    t = t + b1_ref[...]
    t = jnp.where(t >= 0, t, a1_ref[...] * t)
    t = (t * _BN) * g1_ref[...] + be1_ref[...]
    u = lax.dot_general(t, w2_ref[...], (((1,), (1,)), ((), ())),
                        preferred_element_type=jnp.float32)
    u = u + b2_ref[...]
    o_ref[...] = jnp.where(u >= 0, u, a2_ref[...] * u)


_row_spec = pl.BlockSpec((_BM, _D), lambda i: (i, 0))
_par_spec = pl.BlockSpec((_NC, _BM, _D), lambda i: (0, i, 0))
_w_spec = pl.BlockSpec((_D, _D), lambda i: (0, 0))
_v_spec = pl.BlockSpec((1, _D), lambda i: (0, 0))

_mlp_call = pl.pallas_call(
    _mlp_body,
    grid=(_NB,),
    in_specs=[_row_spec, _par_spec, _w_spec, _v_spec, _v_spec, _v_spec,
              _v_spec, _w_spec, _v_spec, _v_spec],
    out_specs=_row_spec,
    out_shape=jax.ShapeDtypeStruct((_N, _D), jnp.float32),
    compiler_params=pltpu.CompilerParams(dimension_semantics=("arbitrary",)),
)


def _fin_body(h_ref, p_ref, w1_ref, b1_ref, a1_ref, g1_ref, be1_ref,
              w2_ref, b2_ref, a2_ref, bat_ref, gf_ref, bf_ref, fw_ref, fb_ref,
              o_ref, acc_ref):
    i = pl.program_id(0)
    z = h_ref[...] + p_ref[0] + p_ref[1]
    t = lax.dot_general(z, w1_ref[...], (((1,), (1,)), ((), ())),
                        preferred_element_type=jnp.float32)
    t = t + b1_ref[...]
    t = jnp.where(t >= 0, t, a1_ref[...] * t)
    t = (t * _BN) * g1_ref[...] + be1_ref[...]
    u = lax.dot_general(t, w2_ref[...], (((1,), (1,)), ((), ())),
                        preferred_element_type=jnp.float32)
    u = u + b2_ref[...]
    u = jnp.where(u >= 0, u, a2_ref[...] * u)

    b = bat_ref[0]
    gi = lax.broadcasted_iota(jnp.int32, (_G, _BM), 0)
    m = (b == gi).astype(jnp.float32)

    @pl.when(i == 0)
    def _():
        acc_ref[...] = jnp.zeros_like(acc_ref)

    acc_ref[...] += jnp.dot(m, u, preferred_element_type=jnp.float32)

    @pl.when(i == _NB - 1)
    def _():
        pooled = (acc_ref[...] * _BN) * gf_ref[...] + bf_ref[...]
        o_ref[...] = lax.dot_general(
            pooled, fw_ref[...], (((1,), (1,)), ((), ())),
            preferred_element_type=jnp.float32) + fb_ref[...]


_fin_call = pl.pallas_call(
    _fin_body,
    grid=(_NB,),
    in_specs=[_row_spec, _par_spec, _w_spec, _v_spec, _v_spec, _v_spec,
              _v_spec, _w_spec, _v_spec, _v_spec,
              pl.BlockSpec((1, 1, _BM), lambda i: (i, 0, 0)),
              pl.BlockSpec((1, _D), lambda i: (0, 0)),
              pl.BlockSpec((1, _D), lambda i: (0, 0)),
              pl.BlockSpec((_L, _D), lambda i: (0, 0)),
              pl.BlockSpec((1, _L), lambda i: (0, 0))],
    out_specs=pl.BlockSpec((_G, _L), lambda i: (0, 0)),
    out_shape=jax.ShapeDtypeStruct((_G, _L), jnp.float32),
    scratch_shapes=[pltpu.VMEM((_G, _D), jnp.float32)],
    compiler_params=pltpu.CompilerParams(dimension_semantics=("arbitrary",)),
)


def kernel(x, edge_index, batch, W1, b1, a1, g1, be1, W2, b2, a2, gf, bf, fcW, fcb):
    bat3 = batch.reshape(_NB, 1, _BM)
    b1r = b1.reshape(_NL, 1, _D)
    a1r = jnp.broadcast_to(a1[:, None, None], (_NL, 1, _D))
    g1r = g1.reshape(_NL, 1, _D)
    be1r = be1.reshape(_NL, 1, _D)
    b2r = b2.reshape(_NL, 1, _D)
    a2r = jnp.broadcast_to(a2[:, None, None], (_NL, 1, _D))
    gfr = gf.reshape(1, _D)
    bfr = bf.reshape(1, _D)
    fbr = fcb.reshape(1, _L)

    zer = jnp.zeros((_NP, _D), jnp.float32)
    h = x
    for i in range(_NL - 1):
        p = _sc_agg(h, edge_index, zer)
        h = _mlp_call(h, p, W1[i], b1r[i], a1r[i], g1r[i], be1r[i],
                      W2[i], b2r[i], a2r[i])
    i = _NL - 1
    p = _sc_agg(h, edge_index, zer)
    return _fin_call(h, p, W1[i], b1r[i], a1r[i], g1r[i], be1r[i],
                     W2[i], b2r[i], a2r[i], bat3, gfr, bfr, fcW, fbr)
